# baseline (speedup 1.0000x reference)
import functools

import jax
import jax.numpy as jnp
from jax import lax
from jax.experimental import pallas as pl
from jax.experimental.pallas import tpu as pltpu
from jax.experimental.pallas import tpu_sc as plsc

_TRAIN_N = 10000
_BS = 4096
_FB = 50
_EMB = 16
_HYPER = 0.01


def _sc_gather(table, idx_flat):
    info = plsc.get_sparse_core_info()
    nw = info.num_cores * info.num_subcores
    b = idx_flat.shape[0]
    b_per_w = b // nw
    rpw = b_per_w // 8
    mesh = plsc.VectorSubcoreMesh(core_axis_name="c", subcore_axis_name="s")

    @functools.partial(
        pl.kernel,
        mesh=mesh,
        out_type=jax.ShapeDtypeStruct((b // 8, 128), jnp.float32),
        scratch_types=[
            pltpu.VMEM((b_per_w,), jnp.int32),
            pltpu.VMEM((b_per_w,), jnp.int32),
            pltpu.VMEM((b_per_w, _EMB), jnp.float32),
            pltpu.SemaphoreType.DMA,
            pltpu.SemaphoreType.DMA,
        ],
        compiler_params=pltpu.CompilerParams(
            use_tc_tiling_on_sc=False, needs_layout_passes=False),
    )
    def sc_kernel(idx_hbm, table_hbm, out_hbm, idx_v, grp_v, rows_v, sem, sem2):
        wid = lax.axis_index("s") * info.num_cores + lax.axis_index("c")
        base = wid * b_per_w
        pltpu.sync_copy(idx_hbm.at[pl.ds(base, b_per_w)], idx_v)
        lanes8 = lax.iota(jnp.int32, _EMB) * 8
        for m in range(8):
            for t in range(rpw // _EMB):
                sel = plsc.load_gather(idx_v, [lanes8 + (8 * _EMB * t + m)])
                grp_v[pl.ds(rpw * m + _EMB * t, _EMB)] = sel
        gdescs = [
            pltpu.async_copy(
                table_hbm.at[grp_v.at[pl.ds(rpw * m, rpw)]],
                rows_v.at[pl.ds(rpw * m, rpw)], sem)
            for m in range(8)
        ]
        for d in gdescs:
            d.wait()
        wdescs = [
            pltpu.async_copy(
                rows_v.at[pl.ds(rpw * m, rpw)],
                out_hbm.at[pl.ds(wid * rpw, rpw), pl.ds(_EMB * m, _EMB)],
                sem2,
            )
            for m in range(8)
        ]
        for d in wdescs:
            d.wait()

    return sc_kernel(idx_flat, table)


def _tc_kernel(gath_ref, embt_ref, w1k_ref, b1r_ref, p23_ref, bsc_ref,
               before_ref, after_ref,
               pred_ref, mse_ref, l2_ref, tot_ref,
               acc_ref):
    j = pl.program_id(0)

    @pl.when(j == 0)
    def _init():
        acc_ref[...] = jnp.zeros_like(acc_ref)

    x = gath_ref[...]
    h = jnp.dot(x, w1k_ref[...], preferred_element_type=jnp.float32) + b1r_ref[...]
    h = jnp.maximum(h, 0.0)
    acc_ref[...] += jnp.dot(h, p23_ref[...], preferred_element_type=jnp.float32)

    @pl.when(j == pl.num_programs(0) - 1)
    def _finish():
        pa = acc_ref[:, 0:8] + _FB * bsc_ref[0]
        pb = acc_ref[:, 8:16] + _FB * bsc_ref[1]
        pred = pa * before_ref[...] + pb
        pred_ref[...] = pred
        diff = after_ref[...] - pred
        mse = jnp.sum(diff * diff) / _BS
        e0 = embt_ref[0:1, :]
        e1 = embt_ref[1:2, :]
        s0 = jnp.sum(e0 * e0)
        s1 = jnp.sum(e1 * e1)
        l2 = _HYPER * (jnp.sqrt(s0) + jnp.sqrt(s1))
        mse_ref[0, 0] = mse
        l2_ref[0, 0] = l2
        tot_ref[0, 0] = mse + l2


def kernel(orders, before_loss, after_loss, test_sample_ids, emb, W1, b1, W2, b2, W3, b3):
    idx_flat = (_TRAIN_N * test_sample_ids[None, :] + orders.T).reshape(-1)
    idx_flat = idx_flat.astype(jnp.int32)

    gath128 = _sc_gather(emb, idx_flat)
    embt = emb.T

    eye8 = jnp.eye(8, dtype=jnp.float32)
    w1k = jnp.kron(eye8, W1)
    b1r = jnp.tile(b1, 8).reshape(1, 800)
    p23 = jnp.concatenate(
        [jnp.kron(eye8, W2), jnp.kron(eye8, W3)], axis=1)
    bsc = jnp.concatenate([b2, b3]).astype(jnp.float32)

    n_steps = _FB
    rows_per_step = gath128.shape[0] // n_steps
    ecols = embt.shape[1]

    out = pl.pallas_call(
        _tc_kernel,
        grid=(n_steps,),
        in_specs=[
            pl.BlockSpec((rows_per_step, 128), lambda j: (j, 0)),
            pl.BlockSpec((8, ecols), lambda j: (0, 0)),
            pl.BlockSpec((128, 800), lambda j: (0, 0)),
            pl.BlockSpec((1, 800), lambda j: (0, 0)),
            pl.BlockSpec((800, 16), lambda j: (0, 0)),
            pl.BlockSpec(memory_space=pltpu.SMEM),
            pl.BlockSpec((rows_per_step, 8), lambda j: (0, 0)),
            pl.BlockSpec((rows_per_step, 8), lambda j: (0, 0)),
        ],
        out_specs=[
            pl.BlockSpec((rows_per_step, 8), lambda j: (0, 0)),
            pl.BlockSpec(memory_space=pltpu.SMEM),
            pl.BlockSpec(memory_space=pltpu.SMEM),
            pl.BlockSpec(memory_space=pltpu.SMEM),
        ],
        out_shape=[
            jax.ShapeDtypeStruct((rows_per_step, 8), jnp.float32),
            jax.ShapeDtypeStruct((1, 1), jnp.float32),
            jax.ShapeDtypeStruct((1, 1), jnp.float32),
            jax.ShapeDtypeStruct((1, 1), jnp.float32),
        ],
        scratch_shapes=[
            pltpu.VMEM((rows_per_step, _EMB), jnp.float32),
        ],
    )(
        gath128, embt, w1k, b1r, p23, bsc,
        before_loss.reshape(rows_per_step, 8), after_loss.reshape(rows_per_step, 8),
    )
    pred2, mse2, l22, tot2 = out
    return (mse2.reshape(()), l22.reshape(()), pred2.reshape(_BS), tot2.reshape(()))


# speedup vs baseline: 1.0249x; 1.0249x over previous
import functools

import jax
import jax.numpy as jnp
from jax import lax
from jax.experimental import pallas as pl
from jax.experimental.pallas import tpu as pltpu
from jax.experimental.pallas import tpu_sc as plsc

_TRAIN_N = 10000
_BS = 4096
_FB = 50
_EMB = 16
_HYPER = 0.01


def _sc_gather(table, idx_flat):
    info = plsc.get_sparse_core_info()
    nw = info.num_cores * info.num_subcores
    b = idx_flat.shape[0]
    b_per_w = b // nw
    rpw = b_per_w // 8
    mesh = plsc.VectorSubcoreMesh(core_axis_name="c", subcore_axis_name="s")

    @functools.partial(
        pl.kernel,
        mesh=mesh,
        out_type=jax.ShapeDtypeStruct((b // 8, 128), jnp.float32),
        scratch_types=[
            pltpu.VMEM((b_per_w,), jnp.int32),
            pltpu.VMEM((b_per_w,), jnp.int32),
            pltpu.VMEM((b_per_w, _EMB), jnp.float32),
            pltpu.SemaphoreType.DMA,
            pltpu.SemaphoreType.DMA,
        ],
        compiler_params=pltpu.CompilerParams(
            use_tc_tiling_on_sc=False, needs_layout_passes=False),
    )
    def sc_kernel(idx_hbm, table_hbm, out_hbm, idx_v, grp_v, rows_v, sem, sem2):
        wid = lax.axis_index("s") * info.num_cores + lax.axis_index("c")
        base = wid * b_per_w
        pltpu.sync_copy(idx_hbm.at[pl.ds(base, b_per_w)], idx_v)
        lanes8 = lax.iota(jnp.int32, _EMB) * 8
        for m in range(8):
            for t in range(rpw // _EMB):
                sel = plsc.load_gather(idx_v, [lanes8 + (8 * _EMB * t + m)])
                grp_v[pl.ds(rpw * m + _EMB * t, _EMB)] = sel
        gdescs = [
            pltpu.async_copy(
                table_hbm.at[grp_v.at[pl.ds(rpw * m, rpw)]],
                rows_v.at[pl.ds(rpw * m, rpw)], sem)
            for m in range(8)
        ]
        for d in gdescs:
            d.wait()
        wdescs = [
            pltpu.async_copy(
                rows_v.at[pl.ds(rpw * m, rpw)],
                out_hbm.at[pl.ds(wid * rpw, rpw), pl.ds(_EMB * m, _EMB)],
                sem2,
            )
            for m in range(8)
        ]
        for d in wdescs:
            d.wait()

    return sc_kernel(idx_flat, table)


def _tc_kernel(gath_ref, embt_ref, w1k_ref, b1r_ref, p23_ref, bsc_ref,
               before_ref, after_ref,
               pred_ref, mse_ref, l2_ref, tot_ref,
               acc_ref):
    j = pl.program_id(0)

    @pl.when(j == 0)
    def _init():
        acc_ref[...] = jnp.zeros_like(acc_ref)

    x = gath_ref[...]
    h = jnp.dot(x, w1k_ref[...], preferred_element_type=jnp.float32) + b1r_ref[...]
    h = jnp.maximum(h, 0.0)
    y = jnp.dot(h, p23_ref[...], preferred_element_type=jnp.float32)
    acc_ref[...] += y[0:512, :] + y[512:1024, :]

    @pl.when(j == pl.num_programs(0) - 1)
    def _finish():
        pa = acc_ref[:, 0:8] + _FB * bsc_ref[0]
        pb = acc_ref[:, 8:16] + _FB * bsc_ref[1]
        pred = pa * before_ref[...] + pb
        pred_ref[...] = pred
        diff = after_ref[...] - pred
        mse = jnp.sum(diff * diff) / _BS
        e0 = embt_ref[0:1, :]
        e1 = embt_ref[1:2, :]
        s0 = jnp.sum(e0 * e0)
        s1 = jnp.sum(e1 * e1)
        l2 = _HYPER * (jnp.sqrt(s0) + jnp.sqrt(s1))
        mse_ref[0, 0] = mse
        l2_ref[0, 0] = l2
        tot_ref[0, 0] = mse + l2


def kernel(orders, before_loss, after_loss, test_sample_ids, emb, W1, b1, W2, b2, W3, b3):
    idx_flat = (_TRAIN_N * test_sample_ids[None, :] + orders.T).reshape(-1)
    idx_flat = idx_flat.astype(jnp.int32)

    gath128 = _sc_gather(emb, idx_flat)
    embt = emb.T

    eye8 = jnp.eye(8, dtype=jnp.float32)
    w1k = jnp.kron(eye8, W1)
    b1r = jnp.tile(b1, 8).reshape(1, 800)
    p23 = jnp.concatenate(
        [jnp.kron(eye8, W2), jnp.kron(eye8, W3)], axis=1)
    bsc = jnp.concatenate([b2, b3]).astype(jnp.float32)

    n_steps = _FB // 2
    rows_per_step = gath128.shape[0] // n_steps
    samp_rows = 512
    ecols = embt.shape[1]

    out = pl.pallas_call(
        _tc_kernel,
        grid=(n_steps,),
        in_specs=[
            pl.BlockSpec((rows_per_step, 128), lambda j: (j, 0)),
            pl.BlockSpec((8, ecols), lambda j: (0, 0)),
            pl.BlockSpec((128, 800), lambda j: (0, 0)),
            pl.BlockSpec((1, 800), lambda j: (0, 0)),
            pl.BlockSpec((800, 16), lambda j: (0, 0)),
            pl.BlockSpec(memory_space=pltpu.SMEM),
            pl.BlockSpec((samp_rows, 8), lambda j: (0, 0)),
            pl.BlockSpec((samp_rows, 8), lambda j: (0, 0)),
        ],
        out_specs=[
            pl.BlockSpec((samp_rows, 8), lambda j: (0, 0)),
            pl.BlockSpec(memory_space=pltpu.SMEM),
            pl.BlockSpec(memory_space=pltpu.SMEM),
            pl.BlockSpec(memory_space=pltpu.SMEM),
        ],
        out_shape=[
            jax.ShapeDtypeStruct((samp_rows, 8), jnp.float32),
            jax.ShapeDtypeStruct((1, 1), jnp.float32),
            jax.ShapeDtypeStruct((1, 1), jnp.float32),
            jax.ShapeDtypeStruct((1, 1), jnp.float32),
        ],
        scratch_shapes=[
            pltpu.VMEM((samp_rows, _EMB), jnp.float32),
        ],
    )(
        gath128, embt, w1k, b1r, p23, bsc,
        before_loss.reshape(samp_rows, 8), after_loss.reshape(samp_rows, 8),
    )
    pred2, mse2, l22, tot2 = out
    return (mse2.reshape(()), l22.reshape(()), pred2.reshape(_BS), tot2.reshape(()))


# speedup vs baseline: 1.0315x; 1.0064x over previous
import functools

import jax
import jax.numpy as jnp
from jax import lax
from jax.experimental import pallas as pl
from jax.experimental.pallas import tpu as pltpu
from jax.experimental.pallas import tpu_sc as plsc

_TRAIN_N = 10000
_BS = 4096
_FB = 50
_EMB = 16
_HYPER = 0.01


def _sc_gather(table, idx_flat):
    info = plsc.get_sparse_core_info()
    nw = info.num_cores * info.num_subcores
    b = idx_flat.shape[0]
    b_per_w = b // nw
    rpw = b_per_w // 8
    mesh = plsc.VectorSubcoreMesh(core_axis_name="c", subcore_axis_name="s")

    @functools.partial(
        pl.kernel,
        mesh=mesh,
        out_type=jax.ShapeDtypeStruct((b // 8, 128), jnp.float32),
        scratch_types=[
            pltpu.VMEM((b_per_w,), jnp.int32),
            pltpu.VMEM((b_per_w,), jnp.int32),
            pltpu.VMEM((b_per_w, _EMB), jnp.float32),
            pltpu.SemaphoreType.DMA,
            pltpu.SemaphoreType.DMA,
        ],
        compiler_params=pltpu.CompilerParams(
            use_tc_tiling_on_sc=False, needs_layout_passes=False),
    )
    def sc_kernel(idx_hbm, table_hbm, out_hbm, idx_v, grp_v, rows_v, sem, sem2):
        wid = lax.axis_index("s") * info.num_cores + lax.axis_index("c")
        base = wid * b_per_w
        pltpu.sync_copy(idx_hbm.at[pl.ds(base, b_per_w)], idx_v)
        lanes8 = lax.iota(jnp.int32, _EMB) * 8
        for m in range(8):
            for t in range(rpw // _EMB):
                sel = plsc.load_gather(idx_v, [lanes8 + (8 * _EMB * t + m)])
                grp_v[pl.ds(rpw * m + _EMB * t, _EMB)] = sel
        gdescs = [
            pltpu.async_copy(
                table_hbm.at[grp_v.at[pl.ds(rpw * m, rpw)]],
                rows_v.at[pl.ds(rpw * m, rpw)], sem)
            for m in range(8)
        ]
        for d in gdescs:
            d.wait()
        wdescs = [
            pltpu.async_copy(
                rows_v.at[pl.ds(rpw * m, rpw)],
                out_hbm.at[pl.ds(wid * rpw, rpw), pl.ds(_EMB * m, _EMB)],
                sem2,
            )
            for m in range(8)
        ]
        for d in wdescs:
            d.wait()

    return sc_kernel(idx_flat, table)


def _tc_kernel(gath_ref, embt_ref, w1k_ref, b1r_ref, p23_ref, bsc_ref,
               before_ref, after_ref,
               pred_ref, mse_ref, l2_ref, tot_ref,
               acc_ref):
    j = pl.program_id(0)

    @pl.when(j == 0)
    def _init():
        acc_ref[...] = jnp.zeros_like(acc_ref)

    x = gath_ref[...]
    h = jnp.dot(x, w1k_ref[...], preferred_element_type=jnp.float32) + b1r_ref[...]
    h = jnp.maximum(h, 0.0)
    y = jnp.dot(h, p23_ref[...], preferred_element_type=jnp.float32)
    folds = y[0:512, :]
    for f in range(1, y.shape[0] // 512):
        folds = folds + y[512 * f:512 * (f + 1), :]
    acc_ref[...] += folds

    @pl.when(j == pl.num_programs(0) - 1)
    def _finish():
        pa = acc_ref[:, 0:8] + _FB * bsc_ref[0]
        pb = acc_ref[:, 8:16] + _FB * bsc_ref[1]
        pred = pa * before_ref[...] + pb
        pred_ref[...] = pred
        diff = after_ref[...] - pred
        mse = jnp.sum(diff * diff) / _BS
        e0 = embt_ref[0:1, :]
        e1 = embt_ref[1:2, :]
        s0 = jnp.sum(e0 * e0)
        s1 = jnp.sum(e1 * e1)
        l2 = _HYPER * (jnp.sqrt(s0) + jnp.sqrt(s1))
        mse_ref[0, 0] = mse
        l2_ref[0, 0] = l2
        tot_ref[0, 0] = mse + l2


def kernel(orders, before_loss, after_loss, test_sample_ids, emb, W1, b1, W2, b2, W3, b3):
    idx_flat = (_TRAIN_N * test_sample_ids[None, :] + orders.T).reshape(-1)
    idx_flat = idx_flat.astype(jnp.int32)

    gath128 = _sc_gather(emb, idx_flat)
    embt = emb.T

    eye8 = jnp.eye(8, dtype=jnp.float32)
    w1k = jnp.kron(eye8, W1)
    b1r = jnp.tile(b1, 8).reshape(1, 800)
    p23 = jnp.concatenate(
        [jnp.kron(eye8, W2), jnp.kron(eye8, W3)], axis=1)
    bsc = jnp.concatenate([b2, b3]).astype(jnp.float32)

    n_steps = _FB // 5
    rows_per_step = gath128.shape[0] // n_steps
    samp_rows = 512
    ecols = embt.shape[1]

    out = pl.pallas_call(
        _tc_kernel,
        grid=(n_steps,),
        in_specs=[
            pl.BlockSpec((rows_per_step, 128), lambda j: (j, 0)),
            pl.BlockSpec((8, ecols), lambda j: (0, 0)),
            pl.BlockSpec((128, 800), lambda j: (0, 0)),
            pl.BlockSpec((1, 800), lambda j: (0, 0)),
            pl.BlockSpec((800, 16), lambda j: (0, 0)),
            pl.BlockSpec(memory_space=pltpu.SMEM),
            pl.BlockSpec((samp_rows, 8), lambda j: (0, 0)),
            pl.BlockSpec((samp_rows, 8), lambda j: (0, 0)),
        ],
        out_specs=[
            pl.BlockSpec((samp_rows, 8), lambda j: (0, 0)),
            pl.BlockSpec(memory_space=pltpu.SMEM),
            pl.BlockSpec(memory_space=pltpu.SMEM),
            pl.BlockSpec(memory_space=pltpu.SMEM),
        ],
        out_shape=[
            jax.ShapeDtypeStruct((samp_rows, 8), jnp.float32),
            jax.ShapeDtypeStruct((1, 1), jnp.float32),
            jax.ShapeDtypeStruct((1, 1), jnp.float32),
            jax.ShapeDtypeStruct((1, 1), jnp.float32),
        ],
        scratch_shapes=[
            pltpu.VMEM((samp_rows, _EMB), jnp.float32),
        ],
    )(
        gath128, embt, w1k, b1r, p23, bsc,
        before_loss.reshape(samp_rows, 8), after_loss.reshape(samp_rows, 8),
    )
    pred2, mse2, l22, tot2 = out
    return (mse2.reshape(()), l22.reshape(()), pred2.reshape(_BS), tot2.reshape(()))
